# baseline (speedup 1.0000x reference)
import functools

import jax
import jax.numpy as jnp
from jax import lax
from jax.experimental import pallas as pl
from jax.experimental.pallas import tpu as pltpu
from jax.experimental.pallas import tpu_sc as plsc

NC = 2
NS = 16
NW = NC * NS

D = 64
BW = 128


def _make_gather(nt, nb):
    assert nb == NW * BW
    mesh = plsc.VectorSubcoreMesh(
        core_axis_name="c", subcore_axis_name="s",
        num_cores=NC, num_subcores=NS)

    @functools.partial(
        pl.kernel,
        mesh=mesh,
        compiler_params=pltpu.CompilerParams(
            use_tc_tiling_on_sc=False, needs_layout_passes=False),
        out_type=jax.ShapeDtypeStruct((nt, D, nb), jnp.float32),
        scratch_types=[
            pltpu.VMEM((nt, BW), jnp.int32),
            pltpu.VMEM((2, BW, D), jnp.float32),
            pltpu.VMEM((2, D, BW), jnp.float32),
            pltpu.SemaphoreType.DMA,
            pltpu.SemaphoreType.DMA,
        ],
    )
    def gather_kernel(idx_hbm, tab_hbm, out_hbm, idx_v, rows_v, tr_v,
                      gsem, osem):
        c = lax.axis_index("c")
        s = lax.axis_index("s")
        wid = s * NC + c
        b0 = wid * BW

        pltpu.sync_copy(idx_hbm.at[:, pl.ds(b0, BW)], idx_v)

        def gather_descr(t, buf):
            return pltpu.make_async_copy(
                tab_hbm.at[idx_v.at[t]], rows_v.at[buf], gsem)

        def out_descr(t, buf):
            return pltpu.make_async_copy(
                tr_v.at[buf], out_hbm.at[t, :, pl.ds(b0, BW)], osem)

        lane = lax.iota(jnp.int32, 16)
        rows16 = [lane + 16 * j for j in range(BW // 16)]

        gather_descr(0, 0).start()

        @pl.loop(0, nt)
        def _(t):
            buf = lax.rem(t, 2)
            gather_descr(t, buf).wait()

            @pl.when(t + 1 < nt)
            def _():
                gather_descr(t + 1, 1 - buf).start()

            @pl.when(t >= 2)
            def _():
                out_descr(t - 2, buf).wait()

            @pl.loop(0, D)
            def _(d):
                dcol = jnp.full((16,), d, jnp.int32)
                for j in range(BW // 16):
                    v = plsc.load_gather(rows_v.at[buf], [rows16[j], dcol])
                    tr_v[buf, d, pl.ds(16 * j, 16)] = v

            out_descr(t, buf).start()

        out_descr(nt - 2, lax.rem(nt, 2)).wait()
        out_descr(nt - 1, lax.rem(nt - 1, 2)).wait()

    return gather_kernel


def kernel(x, weight):
    bsz, t = x.shape
    idx = jnp.transpose(x).astype(jnp.int32)
    out_tdb = _make_gather(t, bsz)(idx, weight)
    return jnp.transpose(out_tdb, (2, 0, 1))


# speedup vs baseline: 1.6467x; 1.6467x over previous
import functools

import jax
import jax.numpy as jnp
from jax import lax
from jax.experimental import pallas as pl
from jax.experimental.pallas import tpu as pltpu
from jax.experimental.pallas import tpu_sc as plsc

NC = 2
NS = 16
NW = NC * NS

D = 64
CHUNK = 512


def _make_gather(B):
    assert B % (NW * CHUNK) == 0
    b_per_w = B // NW
    n_super = b_per_w // CHUNK
    mesh = plsc.VectorSubcoreMesh(
        core_axis_name="c", subcore_axis_name="s",
        num_cores=NC, num_subcores=NS)

    @functools.partial(
        pl.kernel,
        mesh=mesh,
        compiler_params=pltpu.CompilerParams(use_tc_tiling_on_sc=False),
        out_type=jax.ShapeDtypeStruct((B, D), jnp.float32),
        scratch_types=[
            pltpu.VMEM((b_per_w,), jnp.int32),
            pltpu.VMEM((2, CHUNK, D), jnp.float32),
            pltpu.SemaphoreType.DMA,
            pltpu.SemaphoreType.DMA,
        ],
    )
    def gather_kernel(idx_hbm, table_hbm, out_hbm, idx_v, rows_v, gsem, osem):
        c = lax.axis_index("c")
        s = lax.axis_index("s")
        wid = s * NC + c
        base = wid * b_per_w

        pltpu.sync_copy(idx_hbm.at[pl.ds(base, b_per_w)], idx_v)

        def gather_descr(sidx, buf):
            return pltpu.make_async_copy(
                table_hbm.at[idx_v.at[pl.ds(sidx * CHUNK, CHUNK)]],
                rows_v.at[buf],
                gsem)

        def out_descr(sidx, buf):
            return pltpu.make_async_copy(
                rows_v.at[buf],
                out_hbm.at[pl.ds(base + sidx * CHUNK, CHUNK)],
                osem)

        gather_descr(0, 0).start()

        @pl.loop(0, n_super, step=2)
        def _(si):
            for b in range(2):
                sidx = si + b
                gather_descr(sidx, b).wait()

                @pl.when(sidx > 0)
                def _():
                    out_descr(sidx - 1, 1 - b).wait()

                @pl.when(sidx + 1 < n_super)
                def _():
                    gather_descr(sidx + 1, 1 - b).start()

                out_descr(sidx, b).start()

        out_descr(n_super - 1, (n_super - 1) % 2).wait()

    return gather_kernel


def kernel(x, weight):
    B = x.size
    idx = x.reshape(-1).astype(jnp.int32)
    out = _make_gather(B)(idx, weight)
    return out.reshape(*x.shape, D)
